# baseline (speedup 1.0000x reference)
import functools

import jax
import jax.numpy as jnp
from jax import lax
from jax.experimental import pallas as pl
from jax.experimental.pallas import tpu as pltpu
from jax.experimental.pallas import tpu_sc as plsc

NUM_TCRS = 100000
MAX_HLAS = 16
NUM_DONORS = 1024
NUM_HLAS = 200
BATCH = 1024
NUM_POS = 8
BETA = 4.0
L2_LAMBDA = 1e-05

_C_PAD = 256
_NC, _NS = 2, 16
_NW = _NC * _NS
_BPW = BATCH // _NW

_HALF_LOG_2PI = 0.9189385332046727


def _lgamma(x):
    y = x + 8.0
    yi = 1.0 / y
    yi2 = yi * yi
    s = (y - 0.5) * jnp.log(y) - y + _HALF_LOG_2PI
    s = s + yi * (8.333333333333333e-2
                  - yi2 * (2.777777777777778e-3 - yi2 * 7.936507936507937e-4))
    prod = (x * (x + 1.0) * (x + 2.0) * (x + 3.0)
            * (x + 4.0) * (x + 5.0) * (x + 6.0) * (x + 7.0))
    return s - jnp.log(prod)


@functools.cache
def _make_sc_gather():
    mesh = plsc.VectorSubcoreMesh(
        core_axis_name="c", subcore_axis_name="s",
        num_cores=_NC, num_subcores=_NS)

    @functools.partial(
        pl.kernel,
        out_type=(jax.ShapeDtypeStruct((BATCH, MAX_HLAS), jnp.float32),
                  jax.ShapeDtypeStruct((BATCH, MAX_HLAS), jnp.int32)),
        mesh=mesh,
        scratch_types=(
            pltpu.VMEM((_BPW,), jnp.int32),
            pltpu.VMEM((_BPW, MAX_HLAS), jnp.float32),
            pltpu.VMEM((_BPW, MAX_HLAS), jnp.int32),
            pltpu.SemaphoreType.DMA,
            pltpu.SemaphoreType.DMA,
        ),
        compiler_params=pltpu.CompilerParams(use_tc_tiling_on_sc=False),
    )
    def _sc_gather(idx_hbm, z_hbm, bind_hbm, z_out, b_out,
                   idx_v, z_v, b_v, sem_z, sem_b):
        wid = lax.axis_index("s") * _NC + lax.axis_index("c")
        base = wid * _BPW
        pltpu.sync_copy(idx_hbm.at[pl.ds(base, _BPW)], idx_v)
        cz = pltpu.async_copy(z_hbm.at[idx_v], z_v, sem_z)
        cb = pltpu.async_copy(bind_hbm.at[idx_v], b_v, sem_b)
        cz.wait()
        cb.wait()
        pltpu.sync_copy(z_v, z_out.at[pl.ds(base, _BPW)])
        pltpu.sync_copy(b_v, b_out.at[pl.ds(base, _BPW)])

    return _sc_gather


def _tc_body(z_ref, b_ref, xp_ref, pos_ref, nll_ref, reg_ref):
    z = z_ref[...]
    bidx = b_ref[...]
    pos = pos_ref[...]

    m = (bidx != -1).astype(jnp.float32)
    zp = m / (1.0 + jnp.exp(-z))
    w = jnp.log(jnp.maximum(1.0 - zp, 1e-7))

    ccol = lax.broadcasted_iota(jnp.int32, (BATCH, NUM_HLAS), 1)
    s = jnp.zeros((BATCH, NUM_HLAS), jnp.float32)
    for h in range(MAX_HLAS):
        s = s + jnp.where(ccol == bidx[:, h:h + 1], w[:, h:h + 1], 0.0)

    lp = lax.dot_general(s, xp_ref[...], (((1,), (1,)), ((), ())),
                         preferred_element_type=jnp.float32)
    p = jnp.maximum(1.0 - jnp.exp(lp), 1e-7)
    sum_p_all = jnp.sum(p, axis=1, keepdims=True)

    dcol = lax.broadcasted_iota(jnp.int32, (BATCH, NUM_DONORS), 1)
    reward = jnp.zeros((BATCH, 1), jnp.float32)
    sum_p_pos = jnp.zeros((BATCH, 1), jnp.float32)
    n_i = jnp.zeros((BATCH, 1), jnp.float32)
    for j in range(NUM_POS):
        pj = pos[:, j:j + 1]
        mj = (pj != -1).astype(jnp.float32)
        sel = jnp.where(dcol == jnp.maximum(pj, 0), p, 0.0)
        ppj = jnp.sum(sel, axis=1, keepdims=True)
        reward = reward + jnp.log(ppj) * mj
        sum_p_pos = sum_p_pos + ppj * mj
        n_i = n_i + mj

    n_tilde = sum_p_all - sum_p_pos
    pen = _lgamma(n_tilde + BETA) - _lgamma(n_i + n_tilde + BETA + 1.0)
    nll = -jnp.sum(reward + pen)
    n_valid = jnp.maximum(jnp.sum(m), 1.0)
    reg = L2_LAMBDA * jnp.sum(z * z * m) / n_valid
    nll_ref[...] = jnp.reshape(nll, (1, 1))
    reg_ref[...] = jnp.reshape(reg, (1, 1))


def _tc_compute(zrows, brows, xp, pos):
    return pl.pallas_call(
        _tc_body,
        out_shape=(jax.ShapeDtypeStruct((1, 1), jnp.float32),
                   jax.ShapeDtypeStruct((1, 1), jnp.float32)),
    )(zrows, brows, xp, pos)


def kernel(tcr_idx, pos_donor_indices, donor_hla_matrix, binder_sets, z_table):
    zrows, brows = _make_sc_gather()(tcr_idx, z_table, binder_sets)
    nll, reg = _tc_compute(zrows, brows, donor_hla_matrix, pos_donor_indices)
    return (nll[0, 0], reg[0, 0])


# speedup vs baseline: 1.0387x; 1.0387x over previous
import functools

import jax
import jax.numpy as jnp
from jax import lax
from jax.experimental import pallas as pl
from jax.experimental.pallas import tpu as pltpu
from jax.experimental.pallas import tpu_sc as plsc

NUM_TCRS = 100000
MAX_HLAS = 16
NUM_DONORS = 1024
NUM_HLAS = 200
BATCH = 1024
NUM_POS = 8
BETA = 4.0
L2_LAMBDA = 1e-05

_C_PAD = 256
_ZSQ_COL = 208
_M_COL = 224
_NC, _NS = 2, 16
_NW = _NC * _NS
_BPW = BATCH // _NW
_LANES = 128
_RPL = _LANES // MAX_HLAS

_HALF_LOG_2PI = 0.9189385332046727
_LN2 = 0.6931471805599453


def _lgamma(x):
    y = x + 8.0
    yi = 1.0 / y
    yi2 = yi * yi
    s = (y - 0.5) * jnp.log(y) - y + _HALF_LOG_2PI
    s = s + yi * (8.333333333333333e-2
                  - yi2 * (2.777777777777778e-3 - yi2 * 7.936507936507937e-4))
    prod = (x * (x + 1.0) * (x + 2.0) * (x + 3.0)
            * (x + 4.0) * (x + 5.0) * (x + 6.0) * (x + 7.0))
    return s - jnp.log(prod)


def _sc_log(y):
    bits = plsc.bitcast(y, jnp.int32)
    e = lax.shift_right_logical(bits, 23) - 127
    mant = plsc.bitcast((bits & 0x007FFFFF) | 0x3F800000, jnp.float32)
    big = mant > 1.4142135381698608
    mant = jnp.where(big, mant * 0.5, mant)
    e = jnp.where(big, e + 1, e)
    t = mant - 1.0
    s = t / (2.0 + t)
    z2 = s * s
    poly = 1.0 + z2 * (0.33333333 + z2 * (0.2 + z2 * 0.14285715))
    return e.astype(jnp.float32) * _LN2 + 2.0 * s * poly


@functools.cache
def _make_sc_scatter():
    mesh = plsc.VectorSubcoreMesh(
        core_axis_name="c", subcore_axis_name="s",
        num_cores=_NC, num_subcores=_NS)

    @functools.partial(
        pl.kernel,
        out_type=jax.ShapeDtypeStruct((BATCH, _C_PAD), jnp.float32),
        mesh=mesh,
        scratch_types=(
            pltpu.VMEM((_BPW,), jnp.int32),
            pltpu.VMEM((_BPW,), jnp.int32),
            pltpu.VMEM((_BPW, _LANES), jnp.float32),
            pltpu.VMEM((_BPW, _LANES), jnp.int32),
            pltpu.VMEM((_BPW, _C_PAD), jnp.float32),
            pltpu.SemaphoreType.DMA,
            pltpu.SemaphoreType.DMA,
        ),
        compiler_params=pltpu.CompilerParams(needs_layout_passes=False),
    )
    def _sc_scatter(idx_hbm, z2_hbm, b2_hbm, s_out,
                    idx_v, g_v, z_buf, b_buf, s_buf, sem_z, sem_b):
        wid = lax.axis_index("s") * _NC + lax.axis_index("c")
        base = wid * _BPW
        pltpu.sync_copy(idx_hbm.at[pl.ds(base, _BPW)], idx_v)
        for g in range(_BPW // 16):
            g_v[pl.ds(g * 16, 16)] = lax.shift_right_logical(
                idx_v[pl.ds(g * 16, 16)], 3)
        cz = pltpu.async_copy(z2_hbm.at[g_v], z_buf, sem_z)
        cb = pltpu.async_copy(b2_hbm.at[g_v], b_buf, sem_b)
        zeros16 = jnp.zeros((16,), jnp.float32)
        for i in range(_BPW):
            for j in range(_C_PAD // 16):
                s_buf[i, pl.ds(j * 16, 16)] = zeros16
        cz.wait()
        cb.wait()
        lanes = lax.iota(jnp.int32, 16)
        zsq_acc = jnp.zeros((16,), jnp.float32)
        m_acc = jnp.zeros((16,), jnp.float32)
        for g in range(_BPW // 16):
            rows = lanes + g * 16
            off = (idx_v[pl.ds(g * 16, 16)] & (_RPL - 1)) * MAX_HLAS
            for h in range(MAX_HLAS):
                li = off + h
                zc = plsc.load_gather(z_buf, [rows, li])
                bc = plsc.load_gather(b_buf, [rows, li])
                valid = bc != -1
                mf = jnp.where(valid, 1.0, 0.0)
                zp = mf / (1.0 + jnp.exp(-zc))
                w = _sc_log(jnp.maximum(1.0 - zp, 1e-7))
                plsc.addupdate_scatter(s_buf, [rows, bc], w, mask=valid)
                zsq_acc = zsq_acc + zc * zc * mf
                m_acc = m_acc + mf
        s_buf[0, pl.ds(_ZSQ_COL, 16)] = zsq_acc
        s_buf[0, pl.ds(_M_COL, 16)] = m_acc
        pltpu.sync_copy(s_buf, s_out.at[pl.ds(base, _BPW)])

    return _sc_scatter


def _tc_body(s_ref, x_ref, pos_ref, nll_ref, reg_ref):
    s_full = s_ref[...]
    pos = pos_ref[...]

    lp = lax.dot_general(s_full[:, :NUM_HLAS], x_ref[...],
                         (((1,), (1,)), ((), ())),
                         preferred_element_type=jnp.float32)
    p = jnp.maximum(1.0 - jnp.exp(lp), 1e-7)
    sum_p_all = jnp.sum(p, axis=1, keepdims=True)

    dcol = lax.broadcasted_iota(jnp.int32, (BATCH, NUM_DONORS), 1)
    reward = jnp.zeros((BATCH, 1), jnp.float32)
    sum_p_pos = jnp.zeros((BATCH, 1), jnp.float32)
    n_i = jnp.zeros((BATCH, 1), jnp.float32)
    for j in range(NUM_POS):
        pj = pos[:, j:j + 1]
        mj = (pj != -1).astype(jnp.float32)
        sel = jnp.where(dcol == jnp.maximum(pj, 0), p, 0.0)
        ppj = jnp.sum(sel, axis=1, keepdims=True)
        reward = reward + jnp.log(ppj) * mj
        sum_p_pos = sum_p_pos + ppj * mj
        n_i = n_i + mj

    n_tilde = sum_p_all - sum_p_pos
    pen = _lgamma(n_tilde + BETA) - _lgamma(n_i + n_tilde + BETA + 1.0)
    nll = -jnp.sum(reward + pen)
    zsq_total = jnp.sum(s_full[:, _ZSQ_COL:_ZSQ_COL + 16])
    m_total = jnp.sum(s_full[:, _M_COL:_M_COL + 16])
    reg = L2_LAMBDA * zsq_total / jnp.maximum(m_total, 1.0)
    nll_ref[...] = jnp.reshape(nll, (1, 1))
    reg_ref[...] = jnp.reshape(reg, (1, 1))


def _tc_compute(s, x, pos):
    return pl.pallas_call(
        _tc_body,
        out_shape=(jax.ShapeDtypeStruct((1, 1), jnp.float32),
                   jax.ShapeDtypeStruct((1, 1), jnp.float32)),
    )(s, x, pos)


def kernel(tcr_idx, pos_donor_indices, donor_hla_matrix, binder_sets, z_table):
    z2 = z_table.reshape(NUM_TCRS * MAX_HLAS // _LANES, _LANES)
    b2 = binder_sets.reshape(NUM_TCRS * MAX_HLAS // _LANES, _LANES)
    s = _make_sc_scatter()(tcr_idx, z2, b2)
    nll, reg = _tc_compute(s, donor_hla_matrix, pos_donor_indices)
    return (nll[0, 0], reg[0, 0])


# speedup vs baseline: 1.7311x; 1.6666x over previous
import functools

import jax
import jax.numpy as jnp
from jax import lax
from jax.experimental import pallas as pl
from jax.experimental.pallas import tpu as pltpu
from jax.experimental.pallas import tpu_sc as plsc

NUM_TCRS = 100000
MAX_HLAS = 16
NUM_DONORS = 1024
NUM_HLAS = 200
BATCH = 1024
NUM_POS = 8
BETA = 4.0
L2_LAMBDA = 1e-05

_C_PAD = 256
_ZSQ_COL = 208
_M_COL = 224
_NC, _NS = 2, 16
_NW = _NC * _NS
_BPW = BATCH // _NW
_LANES = 128
_RPL = _LANES // MAX_HLAS

_HALF_LOG_2PI = 0.9189385332046727
_LN2 = 0.6931471805599453


def _lgamma(x):
    y = x + 8.0
    yi = 1.0 / y
    yi2 = yi * yi
    s = (y - 0.5) * jnp.log(y) - y + _HALF_LOG_2PI
    s = s + yi * (8.333333333333333e-2
                  - yi2 * (2.777777777777778e-3 - yi2 * 7.936507936507937e-4))
    prod = (x * (x + 1.0) * (x + 2.0) * (x + 3.0)
            * (x + 4.0) * (x + 5.0) * (x + 6.0) * (x + 7.0))
    return s - jnp.log(prod)


def _sc_log(y):
    bits = plsc.bitcast(y, jnp.int32)
    e = lax.shift_right_logical(bits, 23) - 127
    mant = plsc.bitcast((bits & 0x007FFFFF) | 0x3F800000, jnp.float32)
    big = mant > 1.4142135381698608
    mant = jnp.where(big, mant * 0.5, mant)
    e = jnp.where(big, e + 1, e)
    t = mant - 1.0
    s = t / (2.0 + t)
    z2 = s * s
    poly = 1.0 + z2 * (0.33333333 + z2 * (0.2 + z2 * 0.14285715))
    return e.astype(jnp.float32) * _LN2 + 2.0 * s * poly


@functools.cache
def _make_sc_scatter():
    mesh = plsc.VectorSubcoreMesh(
        core_axis_name="c", subcore_axis_name="s",
        num_cores=_NC, num_subcores=_NS)

    @functools.partial(
        pl.kernel,
        out_type=jax.ShapeDtypeStruct((BATCH, _C_PAD), jnp.float32),
        mesh=mesh,
        scratch_types=(
            pltpu.VMEM((_BPW,), jnp.int32),
            pltpu.SMEM((_BPW,), jnp.int32),
            pltpu.VMEM((_BPW, _RPL, MAX_HLAS), jnp.float32),
            pltpu.VMEM((_BPW, _RPL, MAX_HLAS), jnp.int32),
            pltpu.VMEM((_BPW, _C_PAD), jnp.float32),
            pltpu.SemaphoreType.DMA,
            pltpu.SemaphoreType.DMA,
        ),
        compiler_params=pltpu.CompilerParams(needs_layout_passes=False),
    )
    def _sc_scatter(idx_hbm, z2_hbm, b2_hbm, s_out,
                    idx_v, idx_s, z_buf, b_buf, s_buf, sem_z, sem_b):
        wid = lax.axis_index("s") * _NC + lax.axis_index("c")
        base = wid * _BPW
        pltpu.sync_copy(idx_hbm.at[pl.ds(base, _BPW)], idx_v)
        descs = []
        for r in range(_BPW):
            if r % 16 == 0:
                gchunk = lax.shift_right_logical(idx_v[pl.ds(r, 16)], 3)
            g = gchunk[r % 16]
            descs.append(pltpu.async_copy(
                z2_hbm.at[pl.ds(g, 1)], z_buf.at[pl.ds(r, 1)], sem_z))
            descs.append(pltpu.async_copy(
                b2_hbm.at[pl.ds(g, 1)], b_buf.at[pl.ds(r, 1)], sem_b))
        zeros16 = jnp.zeros((16,), jnp.float32)
        for i in range(_BPW):
            for j in range(_C_PAD // 16):
                s_buf[i, pl.ds(j * 16, 16)] = zeros16
        for d in descs:
            d.wait()
        lanes = lax.iota(jnp.int32, 16)
        zsq_acc = jnp.zeros((16,), jnp.float32)
        m_acc = jnp.zeros((16,), jnp.float32)
        for g in range(_BPW // 16):
            rows = lanes + g * 16
            sub = idx_v[pl.ds(g * 16, 16)] & (_RPL - 1)
            for h in range(MAX_HLAS):
                li = jnp.full((16,), h, jnp.int32)
                zc = plsc.load_gather(z_buf, [rows, sub, li])
                bc = plsc.load_gather(b_buf, [rows, sub, li])
                valid = bc != -1
                mf = jnp.where(valid, 1.0, 0.0)
                zp = mf / (1.0 + jnp.exp(-zc))
                w = _sc_log(jnp.maximum(1.0 - zp, 1e-7))
                plsc.addupdate_scatter(s_buf, [rows, bc], w, mask=valid)
                zsq_acc = zsq_acc + zc * zc * mf
                m_acc = m_acc + mf
        s_buf[0, pl.ds(_ZSQ_COL, 16)] = zsq_acc
        s_buf[0, pl.ds(_M_COL, 16)] = m_acc
        pltpu.sync_copy(s_buf, s_out.at[pl.ds(base, _BPW)])

    return _sc_scatter


def _tc_body(s_ref, x_ref, pos_ref, nll_ref, reg_ref):
    s_full = s_ref[...]
    pos = pos_ref[...]

    lp = lax.dot_general(s_full[:, :NUM_HLAS], x_ref[...],
                         (((1,), (1,)), ((), ())),
                         preferred_element_type=jnp.float32)
    p = jnp.maximum(1.0 - jnp.exp(lp), 1e-7)
    sum_p_all = jnp.sum(p, axis=1, keepdims=True)

    dcol = lax.broadcasted_iota(jnp.int32, (BATCH, NUM_DONORS), 1)
    reward = jnp.zeros((BATCH, 1), jnp.float32)
    sum_p_pos = jnp.zeros((BATCH, 1), jnp.float32)
    n_i = jnp.zeros((BATCH, 1), jnp.float32)
    for j in range(NUM_POS):
        pj = pos[:, j:j + 1]
        mj = (pj != -1).astype(jnp.float32)
        sel = jnp.where(dcol == jnp.maximum(pj, 0), p, 0.0)
        ppj = jnp.sum(sel, axis=1, keepdims=True)
        reward = reward + jnp.log(ppj) * mj
        sum_p_pos = sum_p_pos + ppj * mj
        n_i = n_i + mj

    n_tilde = sum_p_all - sum_p_pos
    pen = _lgamma(n_tilde + BETA) - _lgamma(n_i + n_tilde + BETA + 1.0)
    nll = -jnp.sum(reward + pen)
    zsq_total = jnp.sum(s_full[:, _ZSQ_COL:_ZSQ_COL + 16])
    m_total = jnp.sum(s_full[:, _M_COL:_M_COL + 16])
    reg = L2_LAMBDA * zsq_total / jnp.maximum(m_total, 1.0)
    nll_ref[...] = jnp.reshape(nll, (1, 1))
    reg_ref[...] = jnp.reshape(reg, (1, 1))


def _tc_compute(s, x, pos):
    return pl.pallas_call(
        _tc_body,
        out_shape=(jax.ShapeDtypeStruct((1, 1), jnp.float32),
                   jax.ShapeDtypeStruct((1, 1), jnp.float32)),
    )(s, x, pos)


def kernel(tcr_idx, pos_donor_indices, donor_hla_matrix, binder_sets, z_table):
    z2 = z_table.reshape(NUM_TCRS // _RPL, _RPL, MAX_HLAS)
    b2 = binder_sets.reshape(NUM_TCRS // _RPL, _RPL, MAX_HLAS)
    s = _make_sc_scatter()(tcr_idx, z2, b2)
    nll, reg = _tc_compute(s, donor_hla_matrix, pos_donor_indices)
    return (nll[0, 0], reg[0, 0])


# speedup vs baseline: 2.8392x; 1.6401x over previous
import functools

import jax
import jax.numpy as jnp
from jax import lax
from jax.experimental import pallas as pl
from jax.experimental.pallas import tpu as pltpu
from jax.experimental.pallas import tpu_sc as plsc

NUM_TCRS = 100000
MAX_HLAS = 16
NUM_DONORS = 1024
NUM_HLAS = 200
BATCH = 1024
NUM_POS = 8
BETA = 4.0
L2_LAMBDA = 1e-05

_C_PAD = 256
_ZSQ_COL = 208
_M_COL = 224
_NC, _NS = 2, 16
_NW = _NC * _NS
_BPW = BATCH // _NW
_GRP = 16

_HALF_LOG_2PI = 0.9189385332046727
_LN2 = 0.6931471805599453


def _lgamma(x):
    y = x + 8.0
    yi = 1.0 / y
    yi2 = yi * yi
    s = (y - 0.5) * jnp.log(y) - y + _HALF_LOG_2PI
    s = s + yi * (8.333333333333333e-2
                  - yi2 * (2.777777777777778e-3 - yi2 * 7.936507936507937e-4))
    prod = (x * (x + 1.0) * (x + 2.0) * (x + 3.0)
            * (x + 4.0) * (x + 5.0) * (x + 6.0) * (x + 7.0))
    return s - jnp.log(prod)


def _sc_log(y):
    bits = plsc.bitcast(y, jnp.int32)
    e = lax.shift_right_logical(bits, 23) - 127
    mant = plsc.bitcast((bits & 0x007FFFFF) | 0x3F800000, jnp.float32)
    big = mant > 1.4142135381698608
    mant = jnp.where(big, mant * 0.5, mant)
    e = jnp.where(big, e + 1, e)
    t = mant - 1.0
    s = t / (2.0 + t)
    z2 = s * s
    poly = 1.0 + z2 * (0.33333333 + z2 * (0.2 + z2 * 0.14285715))
    return e.astype(jnp.float32) * _LN2 + 2.0 * s * poly


@functools.cache
def _make_sc_scatter():
    mesh = plsc.VectorSubcoreMesh(
        core_axis_name="c", subcore_axis_name="s",
        num_cores=_NC, num_subcores=_NS)

    @functools.partial(
        pl.kernel,
        out_type=jax.ShapeDtypeStruct((BATCH, _C_PAD), jnp.float32),
        mesh=mesh,
        scratch_types=(
            pltpu.VMEM((_BPW,), jnp.int32),
            pltpu.VMEM((_GRP, MAX_HLAS, 128), jnp.float32),
            pltpu.VMEM((_GRP, MAX_HLAS, 128), jnp.int32),
            pltpu.VMEM((_BPW, _C_PAD), jnp.float32),
            pltpu.SemaphoreType.DMA,
            pltpu.SemaphoreType.DMA,
        ),
        compiler_params=pltpu.CompilerParams(needs_layout_passes=False),
    )
    def _sc_scatter(idx_hbm, zt_hbm, bt_hbm, s_out,
                    idx_v, z_blk, b_blk, s_buf, sem_z, sem_b):
        wid = lax.axis_index("s") * _NC + lax.axis_index("c")
        base = wid * _BPW
        pltpu.sync_copy(idx_hbm.at[pl.ds(base, _BPW)], idx_v)
        lanes = lax.iota(jnp.int32, 16)
        zeros16 = jnp.zeros((16,), jnp.float32)
        zsq_acc = jnp.zeros((16,), jnp.float32)
        m_acc = jnp.zeros((16,), jnp.float32)
        for g in range(_BPW // _GRP):
            chunk = idx_v[pl.ds(g * _GRP, _GRP)]
            cbase = lax.shift_left(lax.shift_right_logical(chunk, 7), 7)
            descs = []
            for e in range(_GRP):
                cb = pl.multiple_of(cbase[e], 128)
                descs.append(pltpu.async_copy(
                    zt_hbm.at[:, pl.ds(cb, 128)],
                    z_blk.at[e], sem_z))
                descs.append(pltpu.async_copy(
                    bt_hbm.at[:, pl.ds(cb, 128)],
                    b_blk.at[e], sem_b))
            if g == 0:
                def _zero_row(i):
                    for j in range(_C_PAD // 16):
                        s_buf[i, pl.ds(j * 16, 16)] = zeros16
                pl.loop(0, _BPW)(_zero_row)
            for d in descs:
                d.wait()
            lane_vec = chunk & 127
            rows = lanes + g * _GRP

            def _one_h(h, carry):
                zsq, msum = carry
                hv = jnp.full((16,), h, jnp.int32)
                zc = plsc.load_gather(z_blk, [lanes, hv, lane_vec])
                bc = plsc.load_gather(b_blk, [lanes, hv, lane_vec])
                valid = bc != -1
                mf = jnp.where(valid, 1.0, 0.0)
                zp = mf / (1.0 + jnp.exp(-zc))
                w = _sc_log(jnp.maximum(1.0 - zp, 1e-7))
                plsc.addupdate_scatter(s_buf, [rows, bc], w, mask=valid)
                return zsq + zc * zc * mf, msum + mf

            zsq_acc, m_acc = lax.fori_loop(
                0, MAX_HLAS, _one_h, (zsq_acc, m_acc), unroll=4)
        s_buf[0, pl.ds(_ZSQ_COL, 16)] = zsq_acc
        s_buf[0, pl.ds(_M_COL, 16)] = m_acc
        pltpu.sync_copy(s_buf, s_out.at[pl.ds(base, _BPW)])

    return _sc_scatter


def _tc_body(s_ref, xt_ref, pos_ref, nll_ref, reg_ref):
    s_full = s_ref[...]
    pos = pos_ref[...]

    lp = jnp.dot(s_full[:, :NUM_HLAS], xt_ref[...],
                 preferred_element_type=jnp.float32)
    p = jnp.maximum(1.0 - jnp.exp(lp), 1e-7)
    sum_p_all = jnp.sum(p, axis=1, keepdims=True)

    dcol = lax.broadcasted_iota(jnp.int32, (BATCH, NUM_DONORS), 1)
    cnt = jnp.zeros((BATCH, NUM_DONORS), jnp.float32)
    n_i = jnp.zeros((BATCH, 1), jnp.float32)
    for j in range(NUM_POS):
        pj = pos[:, j:j + 1]
        mj = (pj != -1).astype(jnp.float32)
        enc = jnp.where(pj != -1, pj, NUM_DONORS)
        cnt = cnt + jnp.where(dcol == enc, 1.0, 0.0)
        n_i = n_i + mj
    sum_p_pos = jnp.sum(cnt * p, axis=1, keepdims=True)
    reward = jnp.sum(cnt * jnp.log(p), axis=1, keepdims=True)

    n_tilde = sum_p_all - sum_p_pos
    pen = _lgamma(n_tilde + BETA) - _lgamma(n_i + n_tilde + BETA + 1.0)
    nll = -jnp.sum(reward + pen)
    zsq_total = jnp.sum(s_full[:, _ZSQ_COL:_ZSQ_COL + 16])
    m_total = jnp.sum(s_full[:, _M_COL:_M_COL + 16])
    reg = L2_LAMBDA * zsq_total / jnp.maximum(m_total, 1.0)
    nll_ref[...] = jnp.reshape(nll, (1, 1))
    reg_ref[...] = jnp.reshape(reg, (1, 1))


def _tc_compute(s, xt, pos):
    return pl.pallas_call(
        _tc_body,
        out_shape=(jax.ShapeDtypeStruct((1, 1), jnp.float32),
                   jax.ShapeDtypeStruct((1, 1), jnp.float32)),
    )(s, xt, pos)


def kernel(tcr_idx, pos_donor_indices, donor_hla_matrix, binder_sets, z_table):
    s = _make_sc_scatter()(tcr_idx, z_table.T, binder_sets.T)
    nll, reg = _tc_compute(s, donor_hla_matrix.T, pos_donor_indices)
    return (nll[0, 0], reg[0, 0])
